# initial kernel scaffold (unmeasured)
import jax
import jax.numpy as jnp
from jax import lax
from jax.experimental import pallas as pl
from jax.experimental.pallas import tpu as pltpu

N_DEV = 32
M, N = 4096, 2048
CHUNK = M // N_DEV


def kernel(x, w_mat, scale_x, scale_w):
    def body(x_ref, w_ref, sx_ref, sw_ref, out_ref,
             send_buf, recv_buf, send_sems, recv_sems, credit_sem):
        my = lax.axis_index("i")
        left = (my + N_DEV - 1) % N_DEV
        right = (my + 1) % N_DEV

        scale = sx_ref[0] * sw_ref[0]
        xb = x_ref[:, :].astype(jnp.bfloat16)
        wb = w_ref[:, :].astype(jnp.bfloat16)
        out_ref[:, :] = (
            jnp.dot(xb, wb, preferred_element_type=jnp.float32) * scale
        )

        barrier_sem = pltpu.get_barrier_semaphore()
        for nbr in (left, right):
            pl.semaphore_signal(
                barrier_sem, inc=1,
                device_id=(nbr,), device_id_type=pl.DeviceIdType.MESH,
            )
        pl.semaphore_wait(barrier_sem, 2)

        n_steps = 2 * (N_DEV - 1)
        for s in range(n_steps):
            slot = s % 2
            if s < N_DEV - 1:
                send_chunk = (my - s + 2 * N_DEV) % N_DEV
                recv_chunk = (my - s - 1 + 2 * N_DEV) % N_DEV
            else:
                t = s - (N_DEV - 1)
                send_chunk = (my + 1 - t + 2 * N_DEV) % N_DEV
                recv_chunk = (my - t + 2 * N_DEV) % N_DEV

            send_buf[slot] = out_ref[pl.ds(send_chunk * CHUNK, CHUNK), :]

            if s >= 2:
                pl.semaphore_wait(credit_sem, 1)

            rdma = pltpu.make_async_remote_copy(
                src_ref=send_buf.at[slot],
                dst_ref=recv_buf.at[slot],
                send_sem=send_sems.at[slot],
                recv_sem=recv_sems.at[slot],
                device_id=(right,),
                device_id_type=pl.DeviceIdType.MESH,
            )
            rdma.start()
            rdma.wait()

            if s < N_DEV - 1:
                out_ref[pl.ds(recv_chunk * CHUNK, CHUNK), :] += recv_buf[slot]
            else:
                out_ref[pl.ds(recv_chunk * CHUNK, CHUNK), :] = recv_buf[slot]

            if s < n_steps - 2:
                pl.semaphore_signal(
                    credit_sem, inc=1,
                    device_id=(left,), device_id_type=pl.DeviceIdType.MESH,
                )

    return pl.pallas_call(
        body,
        out_shape=jax.ShapeDtypeStruct((M, N), jnp.float32),
        in_specs=[
            pl.BlockSpec(memory_space=pltpu.VMEM),
            pl.BlockSpec(memory_space=pltpu.VMEM),
            pl.BlockSpec(memory_space=pltpu.SMEM),
            pl.BlockSpec(memory_space=pltpu.SMEM),
        ],
        out_specs=pl.BlockSpec(memory_space=pltpu.VMEM),
        scratch_shapes=[
            pltpu.VMEM((2, CHUNK, N), jnp.float32),
            pltpu.VMEM((2, CHUNK, N), jnp.float32),
            pltpu.SemaphoreType.DMA((2,)),
            pltpu.SemaphoreType.DMA((2,)),
            pltpu.SemaphoreType.REGULAR,
        ],
        compiler_params=pltpu.CompilerParams(collective_id=0),
    )(x, w_mat, scale_x, scale_w)


# baseline (device time: 866802 ns/iter reference)
import jax
import jax.numpy as jnp
from jax import lax
from jax.experimental import pallas as pl
from jax.experimental.pallas import tpu as pltpu

N_DEV = 32
M, N = 4096, 2048
CHUNK = M // N_DEV


def kernel(x, w_mat, scale_x, scale_w):
    def body(x_ref, w_ref, sx_ref, sw_ref, out_ref,
             send_buf, recv_buf, send_sems, recv_sems, credit_sem):
        my = lax.axis_index("i")
        left = (my + N_DEV - 1) % N_DEV
        right = (my + 1) % N_DEV

        scale = sx_ref[0] * sw_ref[0]
        xb = x_ref[:, :].astype(jnp.bfloat16)
        wb = w_ref[:, :].astype(jnp.bfloat16)
        out_ref[:, :] = (
            jnp.dot(xb, wb, preferred_element_type=jnp.float32) * scale
        )

        barrier_sem = pltpu.get_barrier_semaphore()
        for nbr in (left, right):
            pl.semaphore_signal(
                barrier_sem, inc=1,
                device_id=(nbr,), device_id_type=pl.DeviceIdType.MESH,
            )
        pl.semaphore_wait(barrier_sem, 2)

        n_steps = 2 * (N_DEV - 1)
        for s in range(n_steps):
            slot = s % 2
            if s < N_DEV - 1:
                send_chunk = (my - s + 2 * N_DEV) % N_DEV
                recv_chunk = (my - s - 1 + 2 * N_DEV) % N_DEV
            else:
                t = s - (N_DEV - 1)
                send_chunk = (my + 1 - t + 2 * N_DEV) % N_DEV
                recv_chunk = (my - t + 2 * N_DEV) % N_DEV

            send_buf[slot] = out_ref[pl.ds(send_chunk * CHUNK, CHUNK), :]

            if s >= 2:
                pl.semaphore_wait(credit_sem, 1)

            rdma = pltpu.make_async_remote_copy(
                src_ref=send_buf.at[slot],
                dst_ref=recv_buf.at[slot],
                send_sem=send_sems.at[slot],
                recv_sem=recv_sems.at[slot],
                device_id=(right,),
                device_id_type=pl.DeviceIdType.MESH,
            )
            rdma.start()
            rdma.wait()

            if s < N_DEV - 1:
                out_ref[pl.ds(recv_chunk * CHUNK, CHUNK), :] += recv_buf[slot]
            else:
                out_ref[pl.ds(recv_chunk * CHUNK, CHUNK), :] = recv_buf[slot]

            if s < n_steps - 2:
                pl.semaphore_signal(
                    credit_sem, inc=1,
                    device_id=(left,), device_id_type=pl.DeviceIdType.MESH,
                )

    return pl.pallas_call(
        body,
        out_shape=jax.ShapeDtypeStruct((M, N), jnp.float32),
        in_specs=[
            pl.BlockSpec(memory_space=pltpu.VMEM),
            pl.BlockSpec(memory_space=pltpu.VMEM),
            pl.BlockSpec(memory_space=pltpu.SMEM),
            pl.BlockSpec(memory_space=pltpu.SMEM),
        ],
        out_specs=pl.BlockSpec(memory_space=pltpu.VMEM),
        scratch_shapes=[
            pltpu.VMEM((2, CHUNK, N), jnp.float32),
            pltpu.VMEM((2, CHUNK, N), jnp.float32),
            pltpu.SemaphoreType.DMA((2,)),
            pltpu.SemaphoreType.DMA((2,)),
            pltpu.SemaphoreType.REGULAR,
        ],
        compiler_params=pltpu.CompilerParams(
            collective_id=0, vmem_limit_bytes=64 * 1024 * 1024
        ),
    )(x, w_mat, scale_x, scale_w)


# device time: 380613 ns/iter; 2.2774x vs baseline; 2.2774x over previous
import jax
import jax.numpy as jnp
from jax import lax
from jax.experimental import pallas as pl
from jax.experimental.pallas import tpu as pltpu

N_DEV = 32
M, N = 4096, 2048
CK = 128
R_STEPS = 16
L_STEPS = 15
MT = 512


def kernel(x, w_mat, scale_x, scale_w):
    x8 = x.astype(jnp.float8_e4m3fn)
    w8 = w_mat.astype(jnp.float8_e5m2)

    def body(x_ref, w_ref, sx_ref, sw_ref, out_ref,
             xg, wg, wbf,
             xr_send, xr_recv, wr_send, wr_recv,
             xl_send, xl_recv, wl_send, wl_recv,
             creditR, creditL):
        my = lax.axis_index("i")
        left = (my + N_DEV - 1) % N_DEV
        right = (my + 1) % N_DEV

        xg[:, 0:CK] = x_ref[...]
        wg[0:CK, :] = w_ref[...]

        barrier_sem = pltpu.get_barrier_semaphore()
        for nbr in (left, right):
            pl.semaphore_signal(
                barrier_sem, inc=1,
                device_id=(nbr,), device_id_type=pl.DeviceIdType.MESH,
            )
        pl.semaphore_wait(barrier_sem, 2)

        def accum(c0, nc, init=False):
            k0, k1 = c0 * CK, (c0 + nc) * CK
            wbf[0:k1 - k0, :] = wg[k0:k1, :].astype(jnp.bfloat16)
            for mi in range(0, M, MT):
                part = jax.lax.dot_general(
                    xg[mi:mi + MT, k0:k1].astype(jnp.bfloat16),
                    wbf[0:k1 - k0, :],
                    (((1,), (0,)), ((), ())),
                    preferred_element_type=jnp.float32,
                )
                if init:
                    out_ref[mi:mi + MT, :] = part
                else:
                    out_ref[mi:mi + MT, :] += part

        def rdma_pair(src_r, dst_r, xsend, xrecv, wsend, wrecv, slot, dev):
            rx = pltpu.make_async_remote_copy(
                src_ref=xg.at[:, src_r * CK:(src_r + 1) * CK],
                dst_ref=xg.at[:, dst_r * CK:(dst_r + 1) * CK],
                send_sem=xsend.at[slot], recv_sem=xrecv.at[slot],
                device_id=(dev,), device_id_type=pl.DeviceIdType.MESH,
            )
            rw = pltpu.make_async_remote_copy(
                src_ref=wg.at[src_r * CK:(src_r + 1) * CK, :],
                dst_ref=wg.at[dst_r * CK:(dst_r + 1) * CK, :],
                send_sem=wsend.at[slot], recv_sem=wrecv.at[slot],
                device_id=(dev,), device_id_type=pl.DeviceIdType.MESH,
            )
            return rx, rw

        rdR = []
        rdL = []
        for s in range(R_STEPS):
            if s >= 2:
                for r_ in rdR[s - 2]:
                    r_.wait_send()
                pl.semaphore_wait(creditR, 1)
            src_r = 0 if s == 0 else 32 - s
            pair = rdma_pair(src_r, 31 - s,
                             xr_send, xr_recv, wr_send, wr_recv,
                             s % 2, right)
            for r_ in pair:
                r_.start()
            rdR.append(pair)

            if s < L_STEPS:
                if s >= 2:
                    for r_ in rdL[s - 2]:
                        r_.wait_send()
                    pl.semaphore_wait(creditL, 1)
                pair = rdma_pair(s, s + 1,
                                 xl_send, xl_recv, wl_send, wl_recv,
                                 s % 2, left)
                for r_ in pair:
                    r_.start()
                rdL.append(pair)

            if s == 0:
                accum(0, 1, init=True)

            for r_ in rdR[s]:
                r_.wait_recv()
            if s <= R_STEPS - 3:
                pl.semaphore_signal(
                    creditR, inc=1,
                    device_id=(left,), device_id_type=pl.DeviceIdType.MESH,
                )
            if s < L_STEPS:
                for r_ in rdL[s]:
                    r_.wait_recv()
                if s <= L_STEPS - 3:
                    pl.semaphore_signal(
                        creditL, inc=1,
                        device_id=(right,), device_id_type=pl.DeviceIdType.MESH,
                    )

            if s == 3:
                accum(28, 4)
                accum(1, 4)
            elif s == 7:
                accum(24, 4)
                accum(5, 4)
            elif s == 11:
                accum(20, 4)
                accum(9, 4)
            elif s == 14:
                accum(13, 3)
            elif s == 15:
                accum(16, 4)

        for pair in (rdR[R_STEPS - 2], rdR[R_STEPS - 1],
                     rdL[L_STEPS - 2], rdL[L_STEPS - 1]):
            for r_ in pair:
                r_.wait_send()

        scale = sx_ref[0] * sw_ref[0]
        for mi in range(0, M, MT):
            out_ref[mi:mi + MT, :] = out_ref[mi:mi + MT, :] * scale

    return pl.pallas_call(
        body,
        out_shape=jax.ShapeDtypeStruct((M, N), jnp.float32),
        in_specs=[
            pl.BlockSpec(memory_space=pltpu.VMEM),
            pl.BlockSpec(memory_space=pltpu.VMEM),
            pl.BlockSpec(memory_space=pltpu.SMEM),
            pl.BlockSpec(memory_space=pltpu.SMEM),
        ],
        out_specs=pl.BlockSpec(memory_space=pltpu.VMEM),
        scratch_shapes=[
            pltpu.VMEM((M, N_DEV * CK), jnp.float8_e4m3fn),
            pltpu.VMEM((N_DEV * CK, N), jnp.float8_e5m2),
            pltpu.VMEM((4 * CK, N), jnp.bfloat16),
            pltpu.SemaphoreType.DMA((2,)),
            pltpu.SemaphoreType.DMA((2,)),
            pltpu.SemaphoreType.DMA((2,)),
            pltpu.SemaphoreType.DMA((2,)),
            pltpu.SemaphoreType.DMA((2,)),
            pltpu.SemaphoreType.DMA((2,)),
            pltpu.SemaphoreType.DMA((2,)),
            pltpu.SemaphoreType.DMA((2,)),
            pltpu.SemaphoreType.REGULAR,
            pltpu.SemaphoreType.REGULAR,
        ],
        compiler_params=pltpu.CompilerParams(
            collective_id=0, vmem_limit_bytes=64 * 1024 * 1024
        ),
    )(x8, w8, scale_x, scale_w)


# device time: 358925 ns/iter; 2.4150x vs baseline; 1.0604x over previous
import jax
import jax.numpy as jnp
from jax import lax
from jax.experimental import pallas as pl
from jax.experimental.pallas import tpu as pltpu

N_DEV = 32
M, N = 4096, 2048
CK = 128
R_STEPS = 16
L_STEPS = 15
MT = 512


def kernel(x, w_mat, scale_x, scale_w):
    x8 = x.astype(jnp.float8_e4m3fn)
    w8 = w_mat.astype(jnp.float8_e5m2)

    def body(x_ref, w_ref, sx_ref, sw_ref, out_ref,
             xg, wg,
             xr_send, xr_recv, wr_send, wr_recv,
             xl_send, xl_recv, wl_send, wl_recv,
             creditR, creditL):
        my = lax.axis_index("i")
        left = (my + N_DEV - 1) % N_DEV
        right = (my + 1) % N_DEV
        scale = sx_ref[0] * sw_ref[0]

        xg[:, 0:CK] = x_ref[...]
        wg[0:CK, :] = w_ref[...]

        barrier_sem = pltpu.get_barrier_semaphore()
        for nbr in (left, right):
            pl.semaphore_signal(
                barrier_sem, inc=1,
                device_id=(nbr,), device_id_type=pl.DeviceIdType.MESH,
            )
        pl.semaphore_wait(barrier_sem, 2)

        def accum(c0, nc, init=False, fin=False):
            k0, k1 = c0 * CK, (c0 + nc) * CK
            for mi in range(0, M, MT):
                part = jax.lax.dot_general(
                    xg[mi:mi + MT, k0:k1],
                    wg[k0:k1, :],
                    (((1,), (0,)), ((), ())),
                    preferred_element_type=jnp.float32,
                )
                if init:
                    out_ref[mi:mi + MT, :] = part
                elif fin:
                    out_ref[mi:mi + MT, :] = (
                        out_ref[mi:mi + MT, :] + part
                    ) * scale
                else:
                    out_ref[mi:mi + MT, :] += part

        def rdma_pair(src_r, dst_r, xsend, xrecv, wsend, wrecv, slot, dev):
            rx = pltpu.make_async_remote_copy(
                src_ref=xg.at[:, src_r * CK:(src_r + 1) * CK],
                dst_ref=xg.at[:, dst_r * CK:(dst_r + 1) * CK],
                send_sem=xsend.at[slot], recv_sem=xrecv.at[slot],
                device_id=(dev,), device_id_type=pl.DeviceIdType.MESH,
            )
            rw = pltpu.make_async_remote_copy(
                src_ref=wg.at[src_r * CK:(src_r + 1) * CK, :],
                dst_ref=wg.at[dst_r * CK:(dst_r + 1) * CK, :],
                send_sem=wsend.at[slot], recv_sem=wrecv.at[slot],
                device_id=(dev,), device_id_type=pl.DeviceIdType.MESH,
            )
            return rx, rw

        rdR = []
        rdL = []
        pending = []
        for s in range(R_STEPS):
            if s >= 2:
                for r_ in rdR[s - 2]:
                    r_.wait_send()
                pl.semaphore_wait(creditR, 1)
            src_r = 0 if s == 0 else 32 - s
            pair = rdma_pair(src_r, 31 - s,
                             xr_send, xr_recv, wr_send, wr_recv,
                             s % 2, right)
            for r_ in pair:
                r_.start()
            rdR.append(pair)

            if s < L_STEPS:
                if s >= 2:
                    for r_ in rdL[s - 2]:
                        r_.wait_send()
                    pl.semaphore_wait(creditL, 1)
                pair = rdma_pair(s, s + 1,
                                 xl_send, xl_recv, wl_send, wl_recv,
                                 s % 2, left)
                for r_ in pair:
                    r_.start()
                rdL.append(pair)

            if s == 0:
                accum(0, 1, init=True)
            for c0, nc in pending:
                accum(c0, nc)
            pending = []

            for r_ in rdR[s]:
                r_.wait_recv()
            if s <= R_STEPS - 3:
                pl.semaphore_signal(
                    creditR, inc=1,
                    device_id=(left,), device_id_type=pl.DeviceIdType.MESH,
                )
            if s < L_STEPS:
                for r_ in rdL[s]:
                    r_.wait_recv()
                if s <= L_STEPS - 3:
                    pl.semaphore_signal(
                        creditL, inc=1,
                        device_id=(right,), device_id_type=pl.DeviceIdType.MESH,
                    )

            if s == 7:
                pending += [(24, 8), (1, 8)]
            elif s == 14:
                pending += [(9, 7)]

        accum(16, 8, fin=True)

        for pair in (rdR[R_STEPS - 2], rdR[R_STEPS - 1],
                     rdL[L_STEPS - 2], rdL[L_STEPS - 1]):
            for r_ in pair:
                r_.wait_send()

    return pl.pallas_call(
        body,
        out_shape=jax.ShapeDtypeStruct((M, N), jnp.float32),
        in_specs=[
            pl.BlockSpec(memory_space=pltpu.VMEM),
            pl.BlockSpec(memory_space=pltpu.VMEM),
            pl.BlockSpec(memory_space=pltpu.SMEM),
            pl.BlockSpec(memory_space=pltpu.SMEM),
        ],
        out_specs=pl.BlockSpec(memory_space=pltpu.VMEM),
        scratch_shapes=[
            pltpu.VMEM((M, N_DEV * CK), jnp.float8_e4m3fn),
            pltpu.VMEM((N_DEV * CK, N), jnp.float8_e5m2),
            pltpu.SemaphoreType.DMA((2,)),
            pltpu.SemaphoreType.DMA((2,)),
            pltpu.SemaphoreType.DMA((2,)),
            pltpu.SemaphoreType.DMA((2,)),
            pltpu.SemaphoreType.DMA((2,)),
            pltpu.SemaphoreType.DMA((2,)),
            pltpu.SemaphoreType.DMA((2,)),
            pltpu.SemaphoreType.DMA((2,)),
            pltpu.SemaphoreType.REGULAR,
            pltpu.SemaphoreType.REGULAR,
        ],
        compiler_params=pltpu.CompilerParams(
            collective_id=0, vmem_limit_bytes=64 * 1024 * 1024
        ),
    )(x8, w8, scale_x, scale_w)


# device time: 230847 ns/iter; 3.7549x vs baseline; 1.5548x over previous
import jax
import jax.numpy as jnp
from jax import lax
from jax.experimental import pallas as pl
from jax.experimental.pallas import tpu as pltpu

N_DEV = 32
M, N = 4096, 2048
CK = 128
R_STEPS = 16
L_STEPS = 15
MT = 512


def _cycle_tables():
    import distributed_mesh_v7x as dm

    mesh = dm.get_mesh("i", world_size=N_DEV)
    devs = list(mesh.devices)
    coords = [tuple(d.coords) for d in devs]
    logical_of = {c: i for i, c in enumerate(coords)}
    xs = sorted({c[0] for c in coords})
    ys = sorted({c[1] for c in coords})
    zs = sorted({c[2] for c in coords})
    if len(xs) != 2 or len(ys) != 4 or len(zs) != 4:
        succ = [(i + 1) % N_DEV for i in range(N_DEV)]
        pred = [(i - 1) % N_DEV for i in range(N_DEV)]
        return succ, pred
    cycle = []
    for zi, z in enumerate(zs):
        row = ys if zi % 2 == 0 else ys[::-1]
        cycle.extend((xs[0], y, z) for y in row)
    for zi, z in enumerate(zs[::-1]):
        row = ys if zi % 2 == 0 else ys[::-1]
        cycle.extend((xs[1], y, z) for y in row)
    assert len(cycle) == N_DEV and len(set(cycle)) == N_DEV
    for a, b in zip(cycle, cycle[1:] + cycle[:1]):
        assert sum(abs(p - q) for p, q in zip(a, b)) == 1, (a, b)
    succ = [0] * N_DEV
    pred = [0] * N_DEV
    for j, c in enumerate(cycle):
        i = logical_of[c]
        succ[i] = logical_of[cycle[(j + 1) % N_DEV]]
        pred[i] = logical_of[cycle[(j - 1) % N_DEV]]
    return succ, pred


def kernel(x, w_mat, scale_x, scale_w):
    x8 = x.astype(jnp.float8_e4m3fn)
    w8 = w_mat.astype(jnp.float8_e5m2)
    succ_t, pred_t = _cycle_tables()
    my = lax.axis_index("i")
    succ = jnp.asarray(succ_t, jnp.int32)[my][None]
    pred = jnp.asarray(pred_t, jnp.int32)[my][None]

    def body(x_ref, w_ref, sx_ref, sw_ref, succ_ref, pred_ref, out_ref,
             xg, wg,
             xr_send, xr_recv, wr_send, wr_recv,
             xl_send, xl_recv, wl_send, wl_recv,
             creditR, creditL):
        left = pred_ref[0]
        right = succ_ref[0]
        scale = sx_ref[0] * sw_ref[0]

        xg[:, 0:CK] = x_ref[...]
        wg[0:CK, :] = w_ref[...]

        barrier_sem = pltpu.get_barrier_semaphore()
        for nbr in (left, right):
            pl.semaphore_signal(
                barrier_sem, inc=1,
                device_id=(nbr,), device_id_type=pl.DeviceIdType.MESH,
            )
        pl.semaphore_wait(barrier_sem, 2)

        def accum(c0, nc, init=False, fin=False):
            k0, k1 = c0 * CK, (c0 + nc) * CK
            for mi in range(0, M, MT):
                part = jax.lax.dot_general(
                    xg[mi:mi + MT, k0:k1],
                    wg[k0:k1, :],
                    (((1,), (0,)), ((), ())),
                    preferred_element_type=jnp.float32,
                )
                if init:
                    out_ref[mi:mi + MT, :] = part
                elif fin:
                    out_ref[mi:mi + MT, :] = (
                        out_ref[mi:mi + MT, :] + part
                    ) * scale
                else:
                    out_ref[mi:mi + MT, :] += part

        def rdma_pair(src_r, dst_r, xsend, xrecv, wsend, wrecv, slot, dev):
            rx = pltpu.make_async_remote_copy(
                src_ref=xg.at[:, src_r * CK:(src_r + 1) * CK],
                dst_ref=xg.at[:, dst_r * CK:(dst_r + 1) * CK],
                send_sem=xsend.at[slot], recv_sem=xrecv.at[slot],
                device_id=(dev,), device_id_type=pl.DeviceIdType.MESH,
            )
            rw = pltpu.make_async_remote_copy(
                src_ref=wg.at[src_r * CK:(src_r + 1) * CK, :],
                dst_ref=wg.at[dst_r * CK:(dst_r + 1) * CK, :],
                send_sem=wsend.at[slot], recv_sem=wrecv.at[slot],
                device_id=(dev,), device_id_type=pl.DeviceIdType.MESH,
            )
            return rx, rw

        rdR = []
        rdL = []
        pending = []
        for s in range(R_STEPS):
            if s >= 2:
                for r_ in rdR[s - 2]:
                    r_.wait_send()
                pl.semaphore_wait(creditR, 1)
            src_r = 0 if s == 0 else 32 - s
            pair = rdma_pair(src_r, 31 - s,
                             xr_send, xr_recv, wr_send, wr_recv,
                             s % 2, right)
            for r_ in pair:
                r_.start()
            rdR.append(pair)

            if s < L_STEPS:
                if s >= 2:
                    for r_ in rdL[s - 2]:
                        r_.wait_send()
                    pl.semaphore_wait(creditL, 1)
                pair = rdma_pair(s, s + 1,
                                 xl_send, xl_recv, wl_send, wl_recv,
                                 s % 2, left)
                for r_ in pair:
                    r_.start()
                rdL.append(pair)

            if s == 0:
                accum(0, 1, init=True)
            for c0, nc in pending:
                accum(c0, nc)
            pending = []

            for r_ in rdR[s]:
                r_.wait_recv()
            if s <= R_STEPS - 3:
                pl.semaphore_signal(
                    creditR, inc=1,
                    device_id=(left,), device_id_type=pl.DeviceIdType.MESH,
                )
            if s < L_STEPS:
                for r_ in rdL[s]:
                    r_.wait_recv()
                if s <= L_STEPS - 3:
                    pl.semaphore_signal(
                        creditL, inc=1,
                        device_id=(right,), device_id_type=pl.DeviceIdType.MESH,
                    )

            if s == 7:
                pending += [(24, 8), (1, 8)]
            elif s == 14:
                pending += [(9, 7)]

        accum(16, 8, fin=True)

        for pair in (rdR[R_STEPS - 2], rdR[R_STEPS - 1],
                     rdL[L_STEPS - 2], rdL[L_STEPS - 1]):
            for r_ in pair:
                r_.wait_send()

    return pl.pallas_call(
        body,
        out_shape=jax.ShapeDtypeStruct((M, N), jnp.float32),
        in_specs=[
            pl.BlockSpec(memory_space=pltpu.VMEM),
            pl.BlockSpec(memory_space=pltpu.VMEM),
            pl.BlockSpec(memory_space=pltpu.SMEM),
            pl.BlockSpec(memory_space=pltpu.SMEM),
            pl.BlockSpec(memory_space=pltpu.SMEM),
            pl.BlockSpec(memory_space=pltpu.SMEM),
        ],
        out_specs=pl.BlockSpec(memory_space=pltpu.VMEM),
        scratch_shapes=[
            pltpu.VMEM((M, N_DEV * CK), jnp.float8_e4m3fn),
            pltpu.VMEM((N_DEV * CK, N), jnp.float8_e5m2),
            pltpu.SemaphoreType.DMA((2,)),
            pltpu.SemaphoreType.DMA((2,)),
            pltpu.SemaphoreType.DMA((2,)),
            pltpu.SemaphoreType.DMA((2,)),
            pltpu.SemaphoreType.DMA((2,)),
            pltpu.SemaphoreType.DMA((2,)),
            pltpu.SemaphoreType.DMA((2,)),
            pltpu.SemaphoreType.DMA((2,)),
            pltpu.SemaphoreType.REGULAR,
            pltpu.SemaphoreType.REGULAR,
        ],
        compiler_params=pltpu.CompilerParams(
            collective_id=0, vmem_limit_bytes=64 * 1024 * 1024
        ),
    )(x8, w8, scale_x, scale_w, succ, pred)


# device time: 217678 ns/iter; 3.9820x vs baseline; 1.0605x over previous
import jax
import jax.numpy as jnp
from jax import lax
from jax.experimental import pallas as pl
from jax.experimental.pallas import tpu as pltpu

N_DEV = 32
M, N = 4096, 2048
CK = 128
R_STEPS = 16
L_STEPS = 15
MT = 512


def _cycle_tables():
    import distributed_mesh_v7x as dm

    mesh = dm.get_mesh("i", world_size=N_DEV)
    devs = list(mesh.devices)
    coords = [tuple(d.coords) for d in devs]
    logical_of = {c: i for i, c in enumerate(coords)}
    xs = sorted({c[0] for c in coords})
    ys = sorted({c[1] for c in coords})
    zs = sorted({c[2] for c in coords})
    if len(xs) != 2 or len(ys) != 4 or len(zs) != 4:
        succ = [(i + 1) % N_DEV for i in range(N_DEV)]
        pred = [(i - 1) % N_DEV for i in range(N_DEV)]
        return succ, pred
    cycle = []
    for zi, z in enumerate(zs):
        row = ys if zi % 2 == 0 else ys[::-1]
        cycle.extend((xs[0], y, z) for y in row)
    for zi, z in enumerate(zs[::-1]):
        row = ys if zi % 2 == 0 else ys[::-1]
        cycle.extend((xs[1], y, z) for y in row)
    assert len(cycle) == N_DEV and len(set(cycle)) == N_DEV
    for a, b in zip(cycle, cycle[1:] + cycle[:1]):
        assert sum(abs(p - q) for p, q in zip(a, b)) == 1, (a, b)
    succ = [0] * N_DEV
    pred = [0] * N_DEV
    for j, c in enumerate(cycle):
        i = logical_of[c]
        succ[i] = logical_of[cycle[(j + 1) % N_DEV]]
        pred[i] = logical_of[cycle[(j - 1) % N_DEV]]
    return succ, pred


def kernel(x, w_mat, scale_x, scale_w):
    x8 = x.astype(jnp.float8_e4m3fn)
    w8 = w_mat.astype(jnp.float8_e5m2)
    succ_t, pred_t = _cycle_tables()
    my = lax.axis_index("i")
    succ = jnp.asarray(succ_t, jnp.int32)[my][None]
    pred = jnp.asarray(pred_t, jnp.int32)[my][None]

    def body(x_ref, w_ref, sx_ref, sw_ref, succ_ref, pred_ref, out_ref,
             xg, wg,
             xr_send, xr_recv, wr_send, wr_recv,
             xl_send, xl_recv, wl_send, wl_recv,
             creditR, creditL):
        left = pred_ref[0]
        right = succ_ref[0]
        scale = sx_ref[0] * sw_ref[0]

        xg[:, 0:CK] = x_ref[...]
        wg[0:CK, :] = w_ref[...]

        barrier_sem = pltpu.get_barrier_semaphore()
        for nbr in (left, right):
            pl.semaphore_signal(
                barrier_sem, inc=1,
                device_id=(nbr,), device_id_type=pl.DeviceIdType.MESH,
            )
        pl.semaphore_wait(barrier_sem, 2)

        def accum(c0, nc, init=False, fin=False):
            k0, k1 = c0 * CK, (c0 + nc) * CK
            for mi in range(0, M, MT):
                part = jax.lax.dot_general(
                    xg[mi:mi + MT, k0:k1],
                    wg[k0:k1, :],
                    (((1,), (0,)), ((), ())),
                    preferred_element_type=jnp.float32,
                )
                if init:
                    out_ref[mi:mi + MT, :] = part
                elif fin:
                    out_ref[mi:mi + MT, :] = (
                        out_ref[mi:mi + MT, :] + part
                    ) * scale
                else:
                    out_ref[mi:mi + MT, :] += part

        def rdma_pair(src_r, dst_r, xsend, xrecv, wsend, wrecv, slot, dev):
            rx = pltpu.make_async_remote_copy(
                src_ref=xg.at[:, src_r * CK:(src_r + 1) * CK],
                dst_ref=xg.at[:, dst_r * CK:(dst_r + 1) * CK],
                send_sem=xsend.at[slot], recv_sem=xrecv.at[slot],
                device_id=(dev,), device_id_type=pl.DeviceIdType.MESH,
            )
            rw = pltpu.make_async_remote_copy(
                src_ref=wg.at[src_r * CK:(src_r + 1) * CK, :],
                dst_ref=wg.at[dst_r * CK:(dst_r + 1) * CK, :],
                send_sem=wsend.at[slot], recv_sem=wrecv.at[slot],
                device_id=(dev,), device_id_type=pl.DeviceIdType.MESH,
            )
            return rx, rw

        rdR = []
        rdL = []
        pending = []
        for s in range(R_STEPS):
            if s >= 2:
                for r_ in rdR[s - 2]:
                    r_.wait_send()
                pl.semaphore_wait(creditR, 1)
            src_r = 0 if s == 0 else 32 - s
            pair = rdma_pair(src_r, 31 - s,
                             xr_send, xr_recv, wr_send, wr_recv,
                             s % 2, right)
            for r_ in pair:
                r_.start()
            rdR.append(pair)

            if s < L_STEPS:
                if s >= 2:
                    for r_ in rdL[s - 2]:
                        r_.wait_send()
                    pl.semaphore_wait(creditL, 1)
                pair = rdma_pair(s, s + 1,
                                 xl_send, xl_recv, wl_send, wl_recv,
                                 s % 2, left)
                for r_ in pair:
                    r_.start()
                rdL.append(pair)

            if s == 0:
                accum(0, 1, init=True)
            for c0, nc in pending:
                accum(c0, nc)
            pending = []

            for r_ in rdR[s]:
                r_.wait_recv()
            if s <= R_STEPS - 3:
                pl.semaphore_signal(
                    creditR, inc=1,
                    device_id=(left,), device_id_type=pl.DeviceIdType.MESH,
                )
            if s < L_STEPS:
                for r_ in rdL[s]:
                    r_.wait_recv()
                if s <= L_STEPS - 3:
                    pl.semaphore_signal(
                        creditL, inc=1,
                        device_id=(right,), device_id_type=pl.DeviceIdType.MESH,
                    )

            if s == 3:
                pending += [(28, 4), (1, 4)]
            elif s == 7:
                pending += [(24, 4), (5, 4)]
            elif s == 11:
                pending += [(20, 4), (9, 4)]
            elif s == 14:
                pending += [(13, 3)]

        accum(16, 4, fin=True)

        for pair in (rdR[R_STEPS - 2], rdR[R_STEPS - 1],
                     rdL[L_STEPS - 2], rdL[L_STEPS - 1]):
            for r_ in pair:
                r_.wait_send()

    return pl.pallas_call(
        body,
        out_shape=jax.ShapeDtypeStruct((M, N), jnp.float32),
        in_specs=[
            pl.BlockSpec(memory_space=pltpu.VMEM),
            pl.BlockSpec(memory_space=pltpu.VMEM),
            pl.BlockSpec(memory_space=pltpu.SMEM),
            pl.BlockSpec(memory_space=pltpu.SMEM),
            pl.BlockSpec(memory_space=pltpu.SMEM),
            pl.BlockSpec(memory_space=pltpu.SMEM),
        ],
        out_specs=pl.BlockSpec(memory_space=pltpu.VMEM),
        scratch_shapes=[
            pltpu.VMEM((M, N_DEV * CK), jnp.float8_e4m3fn),
            pltpu.VMEM((N_DEV * CK, N), jnp.float8_e5m2),
            pltpu.SemaphoreType.DMA((2,)),
            pltpu.SemaphoreType.DMA((2,)),
            pltpu.SemaphoreType.DMA((2,)),
            pltpu.SemaphoreType.DMA((2,)),
            pltpu.SemaphoreType.DMA((2,)),
            pltpu.SemaphoreType.DMA((2,)),
            pltpu.SemaphoreType.DMA((2,)),
            pltpu.SemaphoreType.DMA((2,)),
            pltpu.SemaphoreType.REGULAR,
            pltpu.SemaphoreType.REGULAR,
        ],
        compiler_params=pltpu.CompilerParams(
            collective_id=0, vmem_limit_bytes=64 * 1024 * 1024
        ),
    )(x8, w8, scale_x, scale_w, succ, pred)


# device time: 217636 ns/iter; 3.9828x vs baseline; 1.0002x over previous
import jax
import jax.numpy as jnp
from jax import lax
from jax.experimental import pallas as pl
from jax.experimental.pallas import tpu as pltpu

N_DEV = 32
M, N = 4096, 2048
CK = 128
R_STEPS = 16
L_STEPS = 15
MT = 512


def _cycle_tables():
    import distributed_mesh_v7x as dm

    mesh = dm.get_mesh("i", world_size=N_DEV)
    devs = list(mesh.devices)
    coords = [tuple(d.coords) for d in devs]
    logical_of = {c: i for i, c in enumerate(coords)}
    xs = sorted({c[0] for c in coords})
    ys = sorted({c[1] for c in coords})
    zs = sorted({c[2] for c in coords})
    if len(xs) != 2 or len(ys) != 4 or len(zs) != 4:
        succ = [(i + 1) % N_DEV for i in range(N_DEV)]
        pred = [(i - 1) % N_DEV for i in range(N_DEV)]
        return succ, pred
    cycle = []
    for zi, z in enumerate(zs):
        row = ys if zi % 2 == 0 else ys[::-1]
        cycle.extend((xs[0], y, z) for y in row)
    for zi, z in enumerate(zs[::-1]):
        row = ys if zi % 2 == 0 else ys[::-1]
        cycle.extend((xs[1], y, z) for y in row)
    assert len(cycle) == N_DEV and len(set(cycle)) == N_DEV
    for a, b in zip(cycle, cycle[1:] + cycle[:1]):
        assert sum(abs(p - q) for p, q in zip(a, b)) == 1, (a, b)
    succ = [0] * N_DEV
    pred = [0] * N_DEV
    for j, c in enumerate(cycle):
        i = logical_of[c]
        succ[i] = logical_of[cycle[(j + 1) % N_DEV]]
        pred[i] = logical_of[cycle[(j - 1) % N_DEV]]
    return succ, pred


def kernel(x, w_mat, scale_x, scale_w):
    x8 = x.astype(jnp.float8_e4m3fn)
    w8 = w_mat.astype(jnp.float8_e5m2)
    succ_t, pred_t = _cycle_tables()
    my = lax.axis_index("i")
    succ = jnp.asarray(succ_t, jnp.int32)[my][None]
    pred = jnp.asarray(pred_t, jnp.int32)[my][None]

    def body(x_ref, w_ref, sx_ref, sw_ref, succ_ref, pred_ref, out_ref,
             xg, wg,
             xr_send, xr_recv, wr_send, wr_recv,
             xl_send, xl_recv, wl_send, wl_recv,
             creditR, creditL):
        left = pred_ref[0]
        right = succ_ref[0]
        scale = sx_ref[0] * sw_ref[0]

        xg[:, 0:CK] = x_ref[...]
        wg[0:CK, :] = w_ref[...]

        barrier_sem = pltpu.get_barrier_semaphore()
        for nbr in (left, right):
            pl.semaphore_signal(
                barrier_sem, inc=1,
                device_id=(nbr,), device_id_type=pl.DeviceIdType.MESH,
            )
        pl.semaphore_wait(barrier_sem, 2)

        def accum(c0, nc, init=False, fin=False):
            k0, k1 = c0 * CK, (c0 + nc) * CK
            for mi in range(0, M, MT):
                part = jax.lax.dot_general(
                    xg[mi:mi + MT, k0:k1],
                    wg[k0:k1, :],
                    (((1,), (0,)), ((), ())),
                    preferred_element_type=jnp.float32,
                )
                if init:
                    out_ref[mi:mi + MT, :] = part
                elif fin:
                    out_ref[mi:mi + MT, :] = (
                        out_ref[mi:mi + MT, :] + part
                    ) * scale
                else:
                    out_ref[mi:mi + MT, :] += part

        def rdma_pair(src_r, dst_r, xsend, xrecv, wsend, wrecv, slot, dev):
            rx = pltpu.make_async_remote_copy(
                src_ref=xg.at[:, src_r * CK:(src_r + 1) * CK],
                dst_ref=xg.at[:, dst_r * CK:(dst_r + 1) * CK],
                send_sem=xsend.at[slot], recv_sem=xrecv.at[slot],
                device_id=(dev,), device_id_type=pl.DeviceIdType.MESH,
            )
            rw = pltpu.make_async_remote_copy(
                src_ref=wg.at[src_r * CK:(src_r + 1) * CK, :],
                dst_ref=wg.at[dst_r * CK:(dst_r + 1) * CK, :],
                send_sem=wsend.at[slot], recv_sem=wrecv.at[slot],
                device_id=(dev,), device_id_type=pl.DeviceIdType.MESH,
            )
            return rx, rw

        rdR = []
        rdL = []
        pending = []
        for s in range(R_STEPS):
            if s >= 3:
                for r_ in rdR[s - 3]:
                    r_.wait_send()
                pl.semaphore_wait(creditR, 1)
            src_r = 0 if s == 0 else 32 - s
            pair = rdma_pair(src_r, 31 - s,
                             xr_send, xr_recv, wr_send, wr_recv,
                             s % 3, right)
            for r_ in pair:
                r_.start()
            rdR.append(pair)

            if s < L_STEPS:
                if s >= 3:
                    for r_ in rdL[s - 3]:
                        r_.wait_send()
                    pl.semaphore_wait(creditL, 1)
                pair = rdma_pair(s, s + 1,
                                 xl_send, xl_recv, wl_send, wl_recv,
                                 s % 3, left)
                for r_ in pair:
                    r_.start()
                rdL.append(pair)

            if s == 0:
                accum(0, 1, init=True)
            for c0, nc in pending:
                accum(c0, nc)
            pending = []

            for r_ in rdR[s]:
                r_.wait_recv()
            if s <= R_STEPS - 4:
                pl.semaphore_signal(
                    creditR, inc=1,
                    device_id=(left,), device_id_type=pl.DeviceIdType.MESH,
                )
            if s < L_STEPS:
                for r_ in rdL[s]:
                    r_.wait_recv()
                if s <= L_STEPS - 4:
                    pl.semaphore_signal(
                        creditL, inc=1,
                        device_id=(right,), device_id_type=pl.DeviceIdType.MESH,
                    )

            if s == 3:
                pending += [(28, 4), (1, 4)]
            elif s == 7:
                pending += [(24, 4), (5, 4)]
            elif s == 11:
                pending += [(20, 4), (9, 4)]
            elif s == 14:
                pending += [(13, 3)]

        accum(16, 4, fin=True)

        for pair in (rdR[R_STEPS - 3], rdR[R_STEPS - 2], rdR[R_STEPS - 1],
                     rdL[L_STEPS - 3], rdL[L_STEPS - 2], rdL[L_STEPS - 1]):
            for r_ in pair:
                r_.wait_send()

    return pl.pallas_call(
        body,
        out_shape=jax.ShapeDtypeStruct((M, N), jnp.float32),
        in_specs=[
            pl.BlockSpec(memory_space=pltpu.VMEM),
            pl.BlockSpec(memory_space=pltpu.VMEM),
            pl.BlockSpec(memory_space=pltpu.SMEM),
            pl.BlockSpec(memory_space=pltpu.SMEM),
            pl.BlockSpec(memory_space=pltpu.SMEM),
            pl.BlockSpec(memory_space=pltpu.SMEM),
        ],
        out_specs=pl.BlockSpec(memory_space=pltpu.VMEM),
        scratch_shapes=[
            pltpu.VMEM((M, N_DEV * CK), jnp.float8_e4m3fn),
            pltpu.VMEM((N_DEV * CK, N), jnp.float8_e5m2),
            pltpu.SemaphoreType.DMA((3,)),
            pltpu.SemaphoreType.DMA((3,)),
            pltpu.SemaphoreType.DMA((3,)),
            pltpu.SemaphoreType.DMA((3,)),
            pltpu.SemaphoreType.DMA((3,)),
            pltpu.SemaphoreType.DMA((3,)),
            pltpu.SemaphoreType.DMA((3,)),
            pltpu.SemaphoreType.DMA((3,)),
            pltpu.SemaphoreType.REGULAR,
            pltpu.SemaphoreType.REGULAR,
        ],
        compiler_params=pltpu.CompilerParams(
            collective_id=0, vmem_limit_bytes=64 * 1024 * 1024
        ),
    )(x8, w8, scale_x, scale_w, succ, pred)


# device time: 210179 ns/iter; 4.1241x vs baseline; 1.0355x over previous
import jax
import jax.numpy as jnp
from jax import lax
from jax.experimental import pallas as pl
from jax.experimental.pallas import tpu as pltpu

N_DEV = 32
M, N = 4096, 2048
CK = 128
R_STEPS = 16
L_STEPS = 15
MT = 512


def _cycle_tables():
    import distributed_mesh_v7x as dm

    mesh = dm.get_mesh("i", world_size=N_DEV)
    devs = list(mesh.devices)
    coords = [tuple(d.coords) for d in devs]
    logical_of = {c: i for i, c in enumerate(coords)}
    xs = sorted({c[0] for c in coords})
    ys = sorted({c[1] for c in coords})
    zs = sorted({c[2] for c in coords})
    if len(xs) != 2 or len(ys) != 4 or len(zs) != 4:
        succ = [(i + 1) % N_DEV for i in range(N_DEV)]
        pred = [(i - 1) % N_DEV for i in range(N_DEV)]
        return succ, pred
    cycle = []
    for zi, z in enumerate(zs):
        row = ys if zi % 2 == 0 else ys[::-1]
        cycle.extend((xs[0], y, z) for y in row)
    for zi, z in enumerate(zs[::-1]):
        row = ys if zi % 2 == 0 else ys[::-1]
        cycle.extend((xs[1], y, z) for y in row)
    assert len(cycle) == N_DEV and len(set(cycle)) == N_DEV
    for a, b in zip(cycle, cycle[1:] + cycle[:1]):
        assert sum(abs(p - q) for p, q in zip(a, b)) == 1, (a, b)
    succ = [0] * N_DEV
    pred = [0] * N_DEV
    for j, c in enumerate(cycle):
        i = logical_of[c]
        succ[i] = logical_of[cycle[(j + 1) % N_DEV]]
        pred[i] = logical_of[cycle[(j - 1) % N_DEV]]
    return succ, pred


def kernel(x, w_mat, scale_x, scale_w):
    x8 = x.astype(jnp.float8_e4m3fn)
    w8 = w_mat.astype(jnp.float8_e5m2)
    succ_t, pred_t = _cycle_tables()
    my = lax.axis_index("i")
    succ = jnp.asarray(succ_t, jnp.int32)[my][None]
    pred = jnp.asarray(pred_t, jnp.int32)[my][None]

    def body(x_ref, w_ref, sx_ref, sw_ref, succ_ref, pred_ref, out_ref,
             xg, wg,
             xr_send, xr_recv, wr_send, wr_recv,
             xl_send, xl_recv, wl_send, wl_recv,
             creditR, creditL):
        left = pred_ref[0]
        right = succ_ref[0]
        scale = sx_ref[0] * sw_ref[0]

        xg[:, 0:CK] = x_ref[...]
        wg[0:CK, :] = w_ref[...]

        barrier_sem = pltpu.get_barrier_semaphore()
        for nbr in (left, right):
            pl.semaphore_signal(
                barrier_sem, inc=1,
                device_id=(nbr,), device_id_type=pl.DeviceIdType.MESH,
            )
        pl.semaphore_wait(barrier_sem, 2)

        def accum(c0, nc, init=False, fin=False):
            k0, k1 = c0 * CK, (c0 + nc) * CK
            for mi in range(0, M, MT):
                part = jax.lax.dot_general(
                    xg[mi:mi + MT, k0:k1],
                    wg[k0:k1, :],
                    (((1,), (0,)), ((), ())),
                    preferred_element_type=jnp.float32,
                )
                if init:
                    out_ref[mi:mi + MT, :] = part
                elif fin:
                    out_ref[mi:mi + MT, :] = (
                        out_ref[mi:mi + MT, :] + part
                    ) * scale
                else:
                    out_ref[mi:mi + MT, :] += part

        def rdma_x(src_r, dst_r, xsend, xrecv, slot, dev):
            return pltpu.make_async_remote_copy(
                src_ref=xg.at[:, src_r * CK:(src_r + 1) * CK],
                dst_ref=xg.at[:, dst_r * CK:(dst_r + 1) * CK],
                send_sem=xsend.at[slot], recv_sem=xrecv.at[slot],
                device_id=(dev,), device_id_type=pl.DeviceIdType.MESH,
            )

        def rdma_w(src_r, dst_r, wsend, wrecv, slot, dev):
            return pltpu.make_async_remote_copy(
                src_ref=wg.at[src_r * CK:(src_r + 1) * CK, :],
                dst_ref=wg.at[dst_r * CK:(dst_r + 1) * CK, :],
                send_sem=wsend.at[slot], recv_sem=wrecv.at[slot],
                device_id=(dev,), device_id_type=pl.DeviceIdType.MESH,
            )

        rdR = []
        rdL = []
        pending = []
        for s in range(R_STEPS + 1):
            if s < R_STEPS and s >= 3:
                for r_ in rdR[s - 3]:
                    r_.wait_send()
                pl.semaphore_wait(creditR, 1)
            if s >= 1:
                rdR[s - 1][0].wait_recv()
            if s < R_STEPS:
                src_r = 0 if s == 0 else 32 - s
                rx = rdma_x(src_r, 31 - s, xr_send, xr_recv, s % 3, right)
                rx.start()
            if s >= 1:
                rdR[s - 1][1].wait_recv()
            if s < R_STEPS:
                rw = rdma_w(src_r, 31 - s, wr_send, wr_recv, s % 3, right)
                rw.start()
                rdR.append((rx, rw))
            if s >= 1 and s - 1 <= R_STEPS - 4:
                pl.semaphore_signal(
                    creditR, inc=1,
                    device_id=(left,), device_id_type=pl.DeviceIdType.MESH,
                )

            if s < L_STEPS and s >= 3:
                for r_ in rdL[s - 3]:
                    r_.wait_send()
                pl.semaphore_wait(creditL, 1)
            if 1 <= s <= L_STEPS:
                rdL[s - 1][0].wait_recv()
            if s < L_STEPS:
                lx = rdma_x(s, s + 1, xl_send, xl_recv, s % 3, left)
                lx.start()
            if 1 <= s <= L_STEPS:
                rdL[s - 1][1].wait_recv()
            if s < L_STEPS:
                lw = rdma_w(s, s + 1, wl_send, wl_recv, s % 3, left)
                lw.start()
                rdL.append((lx, lw))
            if 1 <= s <= L_STEPS and s - 1 <= L_STEPS - 4:
                pl.semaphore_signal(
                    creditL, inc=1,
                    device_id=(right,), device_id_type=pl.DeviceIdType.MESH,
                )

            if s == 0:
                accum(0, 1, init=True)
            for c0, nc in pending:
                accum(c0, nc)
            pending = []
            if s - 1 == 3:
                pending += [(28, 4), (1, 4)]
            elif s - 1 == 7:
                pending += [(24, 4), (5, 4)]
            elif s - 1 == 11:
                pending += [(20, 4), (9, 4)]
            elif s - 1 == 14:
                pending += [(13, 3)]

        accum(16, 4, fin=True)

        for pair in (rdR[R_STEPS - 3], rdR[R_STEPS - 2], rdR[R_STEPS - 1],
                     rdL[L_STEPS - 3], rdL[L_STEPS - 2], rdL[L_STEPS - 1]):
            for r_ in pair:
                r_.wait_send()

    return pl.pallas_call(
        body,
        out_shape=jax.ShapeDtypeStruct((M, N), jnp.float32),
        in_specs=[
            pl.BlockSpec(memory_space=pltpu.VMEM),
            pl.BlockSpec(memory_space=pltpu.VMEM),
            pl.BlockSpec(memory_space=pltpu.SMEM),
            pl.BlockSpec(memory_space=pltpu.SMEM),
            pl.BlockSpec(memory_space=pltpu.SMEM),
            pl.BlockSpec(memory_space=pltpu.SMEM),
        ],
        out_specs=pl.BlockSpec(memory_space=pltpu.VMEM),
        scratch_shapes=[
            pltpu.VMEM((M, N_DEV * CK), jnp.float8_e4m3fn),
            pltpu.VMEM((N_DEV * CK, N), jnp.float8_e5m2),
            pltpu.SemaphoreType.DMA((3,)),
            pltpu.SemaphoreType.DMA((3,)),
            pltpu.SemaphoreType.DMA((3,)),
            pltpu.SemaphoreType.DMA((3,)),
            pltpu.SemaphoreType.DMA((3,)),
            pltpu.SemaphoreType.DMA((3,)),
            pltpu.SemaphoreType.DMA((3,)),
            pltpu.SemaphoreType.DMA((3,)),
            pltpu.SemaphoreType.REGULAR,
            pltpu.SemaphoreType.REGULAR,
        ],
        compiler_params=pltpu.CompilerParams(
            collective_id=0, vmem_limit_bytes=64 * 1024 * 1024
        ),
    )(x8, w8, scale_x, scale_w, succ, pred)


# device time: 209858 ns/iter; 4.1304x vs baseline; 1.0015x over previous
import jax
import jax.numpy as jnp
from jax import lax
from jax.experimental import pallas as pl
from jax.experimental.pallas import tpu as pltpu

N_DEV = 32
M, N = 4096, 2048
CK = 128
R_STEPS = 16
L_STEPS = 15
MT = 512


def _cycle_tables():
    import distributed_mesh_v7x as dm

    mesh = dm.get_mesh("i", world_size=N_DEV)
    devs = list(mesh.devices)
    coords = [tuple(d.coords) for d in devs]
    logical_of = {c: i for i, c in enumerate(coords)}
    xs = sorted({c[0] for c in coords})
    ys = sorted({c[1] for c in coords})
    zs = sorted({c[2] for c in coords})
    if len(xs) != 2 or len(ys) != 4 or len(zs) != 4:
        succ = [(i + 1) % N_DEV for i in range(N_DEV)]
        pred = [(i - 1) % N_DEV for i in range(N_DEV)]
        return succ, pred
    cycle = []
    for zi, z in enumerate(zs):
        row = ys if zi % 2 == 0 else ys[::-1]
        cycle.extend((xs[0], y, z) for y in row)
    for zi, z in enumerate(zs[::-1]):
        row = ys if zi % 2 == 0 else ys[::-1]
        cycle.extend((xs[1], y, z) for y in row)
    assert len(cycle) == N_DEV and len(set(cycle)) == N_DEV
    for a, b in zip(cycle, cycle[1:] + cycle[:1]):
        assert sum(abs(p - q) for p, q in zip(a, b)) == 1, (a, b)
    succ = [0] * N_DEV
    pred = [0] * N_DEV
    for j, c in enumerate(cycle):
        i = logical_of[c]
        succ[i] = logical_of[cycle[(j + 1) % N_DEV]]
        pred[i] = logical_of[cycle[(j - 1) % N_DEV]]
    return succ, pred


def kernel(x, w_mat, scale_x, scale_w):
    x8 = x.astype(jnp.float8_e4m3fn)
    w8 = w_mat.astype(jnp.float8_e5m2)
    succ_t, pred_t = _cycle_tables()
    my = lax.axis_index("i")
    succ = jnp.asarray(succ_t, jnp.int32)[my][None]
    pred = jnp.asarray(pred_t, jnp.int32)[my][None]

    def body(x_ref, w_ref, sx_ref, sw_ref, succ_ref, pred_ref, out_ref,
             xg, wg,
             xr0_send, xr0_recv, xr1_send, xr1_recv, wr_send, wr_recv,
             xl0_send, xl0_recv, xl1_send, xl1_recv, wl_send, wl_recv,
             creditR, creditL):
        left = pred_ref[0]
        right = succ_ref[0]
        scale = sx_ref[0] * sw_ref[0]

        xg[:, 0:CK] = x_ref[...]
        wg[0:CK, :] = w_ref[...]

        barrier_sem = pltpu.get_barrier_semaphore()
        for nbr in (left, right):
            pl.semaphore_signal(
                barrier_sem, inc=1,
                device_id=(nbr,), device_id_type=pl.DeviceIdType.MESH,
            )
        pl.semaphore_wait(barrier_sem, 2)

        def accum(c0, nc, init=False, fin=False):
            k0, k1 = c0 * CK, (c0 + nc) * CK
            for mi in range(0, M, MT):
                part = jax.lax.dot_general(
                    xg[mi:mi + MT, k0:k1],
                    wg[k0:k1, :],
                    (((1,), (0,)), ((), ())),
                    preferred_element_type=jnp.float32,
                )
                if init:
                    out_ref[mi:mi + MT, :] = part
                elif fin:
                    out_ref[mi:mi + MT, :] = (
                        out_ref[mi:mi + MT, :] + part
                    ) * scale
                else:
                    out_ref[mi:mi + MT, :] += part

        def rdma_x(src_r, dst_r, xsend, xrecv, slot, dev, half):
            m0, m1 = half * (M // 2), (half + 1) * (M // 2)
            return pltpu.make_async_remote_copy(
                src_ref=xg.at[m0:m1, src_r * CK:(src_r + 1) * CK],
                dst_ref=xg.at[m0:m1, dst_r * CK:(dst_r + 1) * CK],
                send_sem=xsend.at[slot], recv_sem=xrecv.at[slot],
                device_id=(dev,), device_id_type=pl.DeviceIdType.MESH,
            )

        def rdma_w(src_r, dst_r, wsend, wrecv, slot, dev):
            return pltpu.make_async_remote_copy(
                src_ref=wg.at[src_r * CK:(src_r + 1) * CK, :],
                dst_ref=wg.at[dst_r * CK:(dst_r + 1) * CK, :],
                send_sem=wsend.at[slot], recv_sem=wrecv.at[slot],
                device_id=(dev,), device_id_type=pl.DeviceIdType.MESH,
            )

        rdR = []
        rdL = []
        pending = []
        for s in range(R_STEPS + 1):
            if s < R_STEPS and s >= 3:
                for r_ in rdR[s - 3]:
                    r_.wait_send()
                pl.semaphore_wait(creditR, 1)
            if s >= 1:
                rdR[s - 1][0].wait_recv()
            if s < R_STEPS:
                src_r = 0 if s == 0 else 32 - s
                rx0 = rdma_x(src_r, 31 - s, xr0_send, xr0_recv, s % 3,
                             right, 0)
                rx0.start()
            if s >= 1:
                rdR[s - 1][1].wait_recv()
            if s < R_STEPS:
                rx1 = rdma_x(src_r, 31 - s, xr1_send, xr1_recv, s % 3,
                             right, 1)
                rx1.start()
            if s >= 1:
                rdR[s - 1][2].wait_recv()
            if s < R_STEPS:
                rw = rdma_w(src_r, 31 - s, wr_send, wr_recv, s % 3, right)
                rw.start()
                rdR.append((rx0, rx1, rw))
            if s >= 1 and s - 1 <= R_STEPS - 4:
                pl.semaphore_signal(
                    creditR, inc=1,
                    device_id=(left,), device_id_type=pl.DeviceIdType.MESH,
                )

            if s < L_STEPS and s >= 3:
                for r_ in rdL[s - 3]:
                    r_.wait_send()
                pl.semaphore_wait(creditL, 1)
            if 1 <= s <= L_STEPS:
                rdL[s - 1][0].wait_recv()
            if s < L_STEPS:
                lx0 = rdma_x(s, s + 1, xl0_send, xl0_recv, s % 3, left, 0)
                lx0.start()
            if 1 <= s <= L_STEPS:
                rdL[s - 1][1].wait_recv()
            if s < L_STEPS:
                lx1 = rdma_x(s, s + 1, xl1_send, xl1_recv, s % 3, left, 1)
                lx1.start()
            if 1 <= s <= L_STEPS:
                rdL[s - 1][2].wait_recv()
            if s < L_STEPS:
                lw = rdma_w(s, s + 1, wl_send, wl_recv, s % 3, left)
                lw.start()
                rdL.append((lx0, lx1, lw))
            if 1 <= s <= L_STEPS and s - 1 <= L_STEPS - 4:
                pl.semaphore_signal(
                    creditL, inc=1,
                    device_id=(right,), device_id_type=pl.DeviceIdType.MESH,
                )

            if s == 0:
                accum(0, 1, init=True)
            for c0, nc in pending:
                accum(c0, nc)
            pending = []
            if s - 1 == 3:
                pending += [(28, 4), (1, 4)]
            elif s - 1 == 7:
                pending += [(24, 4), (5, 4)]
            elif s - 1 == 11:
                pending += [(20, 4), (9, 4)]
            elif s - 1 == 14:
                pending += [(13, 3)]

        accum(16, 4, fin=True)

        for pair in (rdR[R_STEPS - 3], rdR[R_STEPS - 2], rdR[R_STEPS - 1],
                     rdL[L_STEPS - 3], rdL[L_STEPS - 2], rdL[L_STEPS - 1]):
            for r_ in pair:
                r_.wait_send()

    return pl.pallas_call(
        body,
        out_shape=jax.ShapeDtypeStruct((M, N), jnp.float32),
        in_specs=[
            pl.BlockSpec(memory_space=pltpu.VMEM),
            pl.BlockSpec(memory_space=pltpu.VMEM),
            pl.BlockSpec(memory_space=pltpu.SMEM),
            pl.BlockSpec(memory_space=pltpu.SMEM),
            pl.BlockSpec(memory_space=pltpu.SMEM),
            pl.BlockSpec(memory_space=pltpu.SMEM),
        ],
        out_specs=pl.BlockSpec(memory_space=pltpu.VMEM),
        scratch_shapes=[
            pltpu.VMEM((M, N_DEV * CK), jnp.float8_e4m3fn),
            pltpu.VMEM((N_DEV * CK, N), jnp.float8_e5m2),
            pltpu.SemaphoreType.DMA((3,)),
            pltpu.SemaphoreType.DMA((3,)),
            pltpu.SemaphoreType.DMA((3,)),
            pltpu.SemaphoreType.DMA((3,)),
            pltpu.SemaphoreType.DMA((3,)),
            pltpu.SemaphoreType.DMA((3,)),
            pltpu.SemaphoreType.DMA((3,)),
            pltpu.SemaphoreType.DMA((3,)),
            pltpu.SemaphoreType.DMA((3,)),
            pltpu.SemaphoreType.DMA((3,)),
            pltpu.SemaphoreType.DMA((3,)),
            pltpu.SemaphoreType.DMA((3,)),
            pltpu.SemaphoreType.REGULAR,
            pltpu.SemaphoreType.REGULAR,
        ],
        compiler_params=pltpu.CompilerParams(
            collective_id=0, vmem_limit_bytes=64 * 1024 * 1024
        ),
    )(x8, w8, scale_x, scale_w, succ, pred)
